# baseline (speedup 1.0000x reference)
import functools

import jax
import jax.numpy as jnp
from jax import lax
from jax.experimental import pallas as pl
from jax.experimental.pallas import tpu as pltpu
from jax.experimental.pallas import tpu_sc as plsc

NC = 2
NS = 16
NW = NC * NS
CHUNK = 128
NPASS = 2
LANES = 16


def _sc_aggregate(feat, src4d, dst4d, n_nodes, d_feat, nchunk, acc_rows):
    zrows = acc_rows // NS
    zchunks = zrows // CHUNK
    zrem = zrows % CHUNK
    pch = nchunk // NPASS

    mesh = plsc.VectorSubcoreMesh(core_axis_name="c", subcore_axis_name="s")

    @functools.partial(
        pl.kernel,
        mesh=mesh,
        out_type=jax.ShapeDtypeStruct((NC, acc_rows, d_feat), jnp.float32),
        scratch_types=[
            pltpu.VMEM((pch, CHUNK), jnp.int32),
            pltpu.VMEM((pch, CHUNK), jnp.int32),
            pltpu.VMEM((10, CHUNK), jnp.int32),
            pltpu.VMEM((CHUNK, d_feat), jnp.float32),
            pltpu.VMEM((CHUNK, d_feat), jnp.float32),
            pltpu.VMEM_SHARED((acc_rows, d_feat), jnp.float32),
            pltpu.SemaphoreType.DMA,
            pltpu.SemaphoreType.DMA,
        ],
    )
    def agg(feat_hbm, src_hbm, dst_hbm, out_hbm, src_v, dst_v, pad_v,
            rows_a, rows_b, acc, sem_a, sem_b):
        rows_v = rows_a
        c = lax.axis_index("c")
        s = lax.axis_index("s")
        wid = c * NS + s

        zbuf = rows_v
        def _zrow(r, carry):
            for k in range(d_feat // LANES):
                zbuf[r, pl.ds(k * LANES, LANES)] = jnp.zeros(
                    (LANES,), jnp.float32)
            return carry
        lax.fori_loop(0, CHUNK, _zrow, 0)
        for z in range(zchunks):
            pltpu.sync_copy(zbuf, acc.at[pl.ds(s * zrows + z * CHUNK, CHUNK)])
        if zrem:
            pltpu.sync_copy(
                zbuf.at[pl.ds(0, zrem)],
                acc.at[pl.ds(s * zrows + zchunks * CHUNK, zrem)])

        plsc.subcore_barrier()

        pad_v[0, pl.ds(0, LANES)] = jnp.zeros((LANES,), jnp.int32)

        def _gather(j, buf, sem):
            pltpu.async_copy(feat_hbm.at[src_v.at[j]], buf, sem)

        def _drain(buf, sem):
            pltpu.make_async_copy(feat_hbm.at[src_v.at[0]], buf, sem).wait()

        def _scatter(j, buf):
            pltpu.sync_copy(buf, acc.at[dst_v.at[j]], add=True)

        for pi in range(NPASS):
            base = pi * pch
            pltpu.sync_copy(src_hbm.at[wid, pl.ds(base, pch)], src_v)
            pltpu.sync_copy(dst_hbm.at[wid, pl.ds(base, pch)], dst_v)

            _gather(0, rows_a, sem_a)

            def _body(p, carry):
                j0 = p * 2
                _drain(rows_a, sem_a)
                _gather(j0 + 1, rows_b, sem_b)
                _scatter(j0, rows_a)
                _drain(rows_b, sem_b)
                _gather(jnp.minimum(j0 + 2, pch - 1), rows_a, sem_a)
                _scatter(j0 + 1, rows_b)
                return carry
            lax.fori_loop(0, pch // 2, _body, 0)
            _drain(rows_a, sem_a)

        plsc.subcore_barrier()

        pltpu.sync_copy(acc.at[pl.ds(s * zrows, zrows)],
                        out_hbm.at[c, pl.ds(s * zrows, zrows)])

    return agg(feat, src4d, dst4d)


def _combine(feat, partials, eps, n_nodes, d_feat):
    blocks = 10
    rows = n_nodes // blocks

    def body(eps_ref, feat_ref, p_ref, out_ref):
        out_ref[...] = ((1.0 + eps_ref[0]) * feat_ref[...]
                        + p_ref[0] + p_ref[1])

    return pl.pallas_call(
        body,
        grid=(blocks,),
        in_specs=[
            pl.BlockSpec(memory_space=pltpu.SMEM),
            pl.BlockSpec((rows, d_feat), lambda i: (i, 0)),
            pl.BlockSpec((NC, rows, d_feat), lambda i: (0, i, 0)),
        ],
        out_specs=pl.BlockSpec((rows, d_feat), lambda i: (i, 0)),
        out_shape=jax.ShapeDtypeStruct((n_nodes, d_feat), jnp.float32),
    )(eps, feat, partials)


def kernel(feat, edge_index, eps):
    n_nodes, d_feat = feat.shape
    n_edges = edge_index.shape[1]

    quant = 2 * NPASS
    nchunk = -(-n_edges // (NW * CHUNK))
    nchunk = -(-nchunk // quant) * quant
    epad = NW * nchunk * CHUNK
    acc_rows = -(-(n_nodes + 1) // (NS * 8)) * NS * 8

    src = edge_index[0]
    dst = edge_index[1]
    pad = epad - n_edges
    src_p = jnp.concatenate([src, jnp.zeros((pad,), jnp.int32)])
    dst_p = jnp.concatenate([dst, jnp.full((pad,), n_nodes, jnp.int32)])
    src3d = src_p.reshape(NW, nchunk, CHUNK)
    dst3d = dst_p.reshape(NW, nchunk, CHUNK)

    partials = _sc_aggregate(feat, src3d, dst3d, n_nodes, d_feat,
                             nchunk, acc_rows)
    return _combine(feat, partials, eps, n_nodes, d_feat)


# speedup vs baseline: 1.4730x; 1.4730x over previous
import functools

import jax
import jax.numpy as jnp
from jax import lax
from jax.experimental import pallas as pl
from jax.experimental.pallas import tpu as pltpu
from jax.experimental.pallas import tpu_sc as plsc

NC = 2
NS = 16
NW = NC * NS
CHUNK = 128
LANES = 16


def _sc_aggregate(feat, src3d, dst3d, n_nodes, d_feat, nchunk, acc_rows):
    zrows = acc_rows // NS
    zchunks = zrows // CHUNK
    zrem = zrows % CHUNK

    mesh = plsc.VectorSubcoreMesh(core_axis_name="c", subcore_axis_name="s")

    @functools.partial(
        pl.kernel,
        mesh=mesh,
        out_type=jax.ShapeDtypeStruct((NC, acc_rows, d_feat), jnp.float32),
        scratch_types=[
            pltpu.VMEM((nchunk, CHUNK), jnp.int32),
            pltpu.VMEM((nchunk, CHUNK), jnp.int32),
            pltpu.VMEM((CHUNK, d_feat), jnp.float32),
            pltpu.VMEM_SHARED((acc_rows, d_feat), jnp.float32),
            pltpu.SemaphoreType.DMA,
        ],
    )
    def agg(feat_hbm, src_hbm, dst_hbm, out_hbm, src_v, dst_v, rows_v,
            acc, sem):
        c = lax.axis_index("c")
        s = lax.axis_index("s")
        wid = c * NS + s

        def _zrow(r, carry):
            for k in range(d_feat // LANES):
                rows_v[r, pl.ds(k * LANES, LANES)] = jnp.zeros(
                    (LANES,), jnp.float32)
            return carry
        lax.fori_loop(0, CHUNK, _zrow, 0)
        for z in range(zchunks):
            pltpu.sync_copy(rows_v, acc.at[pl.ds(s * zrows + z * CHUNK, CHUNK)])
        if zrem:
            pltpu.sync_copy(
                rows_v.at[pl.ds(0, zrem)],
                acc.at[pl.ds(s * zrows + zchunks * CHUNK, zrem)])

        pltpu.sync_copy(src_hbm.at[wid], src_v)
        pltpu.sync_copy(dst_hbm.at[wid], dst_v)

        plsc.subcore_barrier()

        def _body(j, carry):
            pltpu.async_copy(feat_hbm.at[src_v.at[j]], rows_v, sem).wait()
            pltpu.sync_copy(rows_v, acc.at[dst_v.at[j]], add=True)
            return carry
        lax.fori_loop(0, nchunk, _body, 0)

        plsc.subcore_barrier()

        pltpu.sync_copy(acc.at[pl.ds(s * zrows, zrows)],
                        out_hbm.at[c, pl.ds(s * zrows, zrows)])

    return agg(feat, src3d, dst3d)


def _combine(feat, partials, eps, n_nodes, d_feat):
    blocks = 10
    rows = n_nodes // blocks

    def body(eps_ref, feat_ref, p_ref, out_ref):
        out_ref[...] = ((1.0 + eps_ref[0]) * feat_ref[...]
                        + p_ref[0] + p_ref[1])

    return pl.pallas_call(
        body,
        grid=(blocks,),
        in_specs=[
            pl.BlockSpec(memory_space=pltpu.SMEM),
            pl.BlockSpec((rows, d_feat), lambda i: (i, 0)),
            pl.BlockSpec((NC, rows, d_feat), lambda i: (0, i, 0)),
        ],
        out_specs=pl.BlockSpec((rows, d_feat), lambda i: (i, 0)),
        out_shape=jax.ShapeDtypeStruct((n_nodes, d_feat), jnp.float32),
    )(eps, feat, partials)


def kernel(feat, edge_index, eps):
    n_nodes, d_feat = feat.shape
    n_edges = edge_index.shape[1]

    nchunk = -(-n_edges // (NW * CHUNK))
    epad = NW * nchunk * CHUNK
    acc_rows = -(-(n_nodes + 1) // (NS * 8)) * NS * 8

    src = edge_index[0]
    dst = edge_index[1]
    pad = epad - n_edges
    src_p = jnp.concatenate([src, jnp.zeros((pad,), jnp.int32)])
    dst_p = jnp.concatenate([dst, jnp.full((pad,), n_nodes, jnp.int32)])
    src3d = src_p.reshape(NW, nchunk, CHUNK)
    dst3d = dst_p.reshape(NW, nchunk, CHUNK)

    partials = _sc_aggregate(feat, src3d, dst3d, n_nodes, d_feat,
                             nchunk, acc_rows)
    return _combine(feat, partials, eps, n_nodes, d_feat)
